# baseline (speedup 1.0000x reference)
import math

import jax
import jax.numpy as jnp
from jax.experimental import pallas as pl
from jax.experimental.pallas import tpu as pltpu

_K = 5
_EPS = 1e-12
_BM = 256
_LANES = 128


def _body(xb_ref, yb_ref, xa_ref, ya_ref, out_ref, c0_s, sqx_s, sqy_s):
    f32 = jnp.float32
    bf16 = jnp.bfloat16
    dn = (((1,), (1,)), ((), ()))
    n = xa_ref.shape[0]

    @pl.when(pl.program_id(0) == 0)
    def _init():
        Xa = xa_ref[...]
        Ya = ya_ref[...]
        ones_x = jnp.ones((1, Xa.shape[1]), f32)
        ones_y = jnp.ones((1, Ya.shape[1]), f32)
        sqx_row = jax.lax.dot_general(ones_x, Xa * Xa, dn,
                                      preferred_element_type=f32)
        sqy_row = jax.lax.dot_general(ones_y, Ya * Ya, dn,
                                      preferred_element_type=f32)
        sqx_s[...] = sqx_row
        sqy_s[...] = sqy_row
        x0 = Xa[0:1, :]
        y0 = Ya[0:1, :]
        s0x = jnp.sum(x0 * x0, axis=1, keepdims=True)
        s0y = jnp.sum(y0 * y0, axis=1, keepdims=True)
        g0x = jax.lax.dot_general(x0, Xa, dn,
                                  preferred_element_type=f32)
        g0y = jax.lax.dot_general(y0, Ya, dn,
                                  preferred_element_type=f32)
        c0x = jnp.maximum(sqx_row + s0x - 2.0 * g0x, _EPS)
        c0y = jnp.maximum(sqy_row + s0y - 2.0 * g0y, _EPS)
        c0xh = c0x.astype(bf16)
        c0yh = c0y.astype(bf16)
        c0xl = (c0x - c0xh.astype(f32)).astype(bf16)
        c0yl = (c0y - c0yh.astype(f32)).astype(bf16)
        c0_s[...] = jnp.concatenate((c0xh, c0yh, c0xl, c0yl), axis=0)

    Xb = xb_ref[...]
    Yb = yb_ref[...]
    bm = Xb.shape[0]
    sqx_row = sqx_s[...]
    sqy_row = sqy_s[...]

    sqx_b = jnp.sum(Xb * Xb, axis=1, keepdims=True)
    sqy_b = jnp.sum(Yb * Yb, axis=1, keepdims=True)

    Gxm = jax.lax.dot_general(-2.0 * Xb, xa_ref[...], dn,
                              preferred_element_type=f32)
    Gym = jax.lax.dot_general(-2.0 * Yb, ya_ref[...], dn,
                              preferred_element_type=f32)

    ux = sqx_row + Gxm
    uy = sqy_row + Gym
    uw = ux + uy

    inf = jnp.full((bm, _LANES), jnp.inf, f32)
    m1, m2 = inf, inf
    for t in range(n // _LANES):
        s = uw[:, t * _LANES:(t + 1) * _LANES]
        hi = jnp.maximum(m1, s)
        m1 = jnp.minimum(m1, s)
        m2 = jnp.minimum(m2, hi)

    v = jnp.min(jnp.minimum(m1, m2), axis=1, keepdims=True)
    for _ in range(_K):
        has1 = m1 == v
        m1 = jnp.where(has1, m2, m1)
        m2 = jnp.where(has1, jnp.inf, m2)
        v = jnp.min(jnp.minimum(m1, m2), axis=1, keepdims=True)

    eqf = (uw == v).astype(bf16)
    a2p = jax.lax.dot_general(eqf, c0_s[...], dn,
                              preferred_element_type=f32)
    a2x = a2p[:, 0:1] + a2p[:, 2:3]
    a2y = a2p[:, 1:2] + a2p[:, 3:4]

    indx = (ux < (a2x - sqx_b)).astype(bf16)
    indy = (uy < (a2y - sqy_b)).astype(bf16)
    ones_n = jnp.ones((n, 1), bf16)
    mm = (((1,), (0,)), ((), ()))
    cnt = jax.lax.dot_general(jnp.concatenate((indx, indy), axis=0),
                              ones_n, mm, preferred_element_type=f32)
    nx = jnp.where(a2x > _EPS, cnt[:bm], 0.0)
    ny = jnp.where(a2y > _EPS, cnt[bm:], 0.0)

    part = jnp.sum(jnp.log(nx + 1e-7) + jnp.log(ny + 1e-7))
    out_ref[...] = jnp.reshape(part, (1, 1, 1))


def _run(X, y, interpret=False):
    n, dx = X.shape
    dy = y.shape[1]
    grid = n // _BM
    parts = pl.pallas_call(
        _body,
        grid=(grid,),
        in_specs=[
            pl.BlockSpec((_BM, dx), lambda i: (i, 0)),
            pl.BlockSpec((_BM, dy), lambda i: (i, 0)),
            pl.BlockSpec((n, dx), lambda i: (0, 0)),
            pl.BlockSpec((n, dy), lambda i: (0, 0)),
        ],
        out_specs=pl.BlockSpec((1, 1, 1), lambda i: (i, 0, 0)),
        out_shape=jax.ShapeDtypeStruct((grid, 1, 1), jnp.float32),
        scratch_shapes=[
            pltpu.VMEM((4, n), jnp.bfloat16),
            pltpu.VMEM((1, n), jnp.float32),
            pltpu.VMEM((1, n), jnp.float32),
        ],
        interpret=interpret,
    )(X, y, X, y)

    cx = math.pi ** (dx / 2.0) / math.gamma(dx / 2.0 + 1)
    cy = math.pi ** (dy / 2.0) / math.gamma(dy / 2.0 + 1)
    cxy = math.pi ** ((dx + dy) / 2.0) / math.gamma((dx + dy) / 2.0 + 1)
    c_log = math.log(cx * cy / cxy)
    digamma_k = -0.5772156649015329 + sum(1.0 / j for j in range(1, _K))
    n_avg_log = jnp.sum(parts) / jnp.float32(n)
    mi = (jnp.log(jnp.float32(n)) + jnp.float32(c_log)
          + jnp.float32(digamma_k) - n_avg_log) / jnp.log(jnp.float32(2.0))
    return jax.nn.relu(mi)


def kernel(X, y):
    return _run(X, y)


# speedup vs baseline: 1.2996x; 1.2996x over previous
import math

import jax
import jax.numpy as jnp
from jax.experimental import pallas as pl
from jax.experimental.pallas import tpu as pltpu

_K = 5
_EPS = 1e-12
_BM = 256
_LANES = 128


def _body(xb_ref, yb_ref, xa_ref, ya_ref, out_ref, c0_s, sqx_s, sqy_s):
    f32 = jnp.float32
    bf16 = jnp.bfloat16
    dn = (((1,), (1,)), ((), ()))
    n = xa_ref.shape[0]

    @pl.when(pl.program_id(0) == 0)
    def _init():
        Xa = xa_ref[...]
        Ya = ya_ref[...]
        ones_x = jnp.ones((1, Xa.shape[1]), f32)
        ones_y = jnp.ones((1, Ya.shape[1]), f32)
        sqx_s[...] = jax.lax.dot_general(ones_x, Xa * Xa, dn,
                                         preferred_element_type=f32)
        sqy_s[...] = jax.lax.dot_general(ones_y, Ya * Ya, dn,
                                         preferred_element_type=f32)
        sqx_col = jnp.sum(Xa * Xa, axis=1, keepdims=True)
        sqy_col = jnp.sum(Ya * Ya, axis=1, keepdims=True)
        x0 = Xa[0:1, :]
        y0 = Ya[0:1, :]
        s0x = jnp.sum(x0 * x0, axis=1, keepdims=True)
        s0y = jnp.sum(y0 * y0, axis=1, keepdims=True)
        g0x = jax.lax.dot_general(Xa, x0, dn,
                                  preferred_element_type=f32)
        g0y = jax.lax.dot_general(Ya, y0, dn,
                                  preferred_element_type=f32)
        c0x = jnp.maximum(sqx_col + s0x - 2.0 * g0x, _EPS)
        c0y = jnp.maximum(sqy_col + s0y - 2.0 * g0y, _EPS)
        c0xh = c0x.astype(bf16)
        c0yh = c0y.astype(bf16)
        c0xl = (c0x - c0xh.astype(f32)).astype(bf16)
        c0yl = (c0y - c0yh.astype(f32)).astype(bf16)
        c0_s[...] = jnp.concatenate((c0xh, c0yh, c0xl, c0yl), axis=1)

    Xb = xb_ref[...]
    Yb = yb_ref[...]
    bm = Xb.shape[0]
    sqx_row = sqx_s[...]
    sqy_row = sqy_s[...]

    sqx_b = jnp.sum(Xb * Xb, axis=1, keepdims=True)
    sqy_b = jnp.sum(Yb * Yb, axis=1, keepdims=True)

    Gxm = jax.lax.dot_general(-2.0 * Xb, xa_ref[...], dn,
                              preferred_element_type=f32)
    Gym = jax.lax.dot_general(-2.0 * Yb, ya_ref[...], dn,
                              preferred_element_type=f32)

    ux = sqx_row + Gxm
    uy = sqy_row + Gym
    uw = ux + uy

    inf = jnp.full((bm, _LANES), jnp.inf, f32)
    m1, m2 = inf, inf
    for t in range(n // _LANES):
        s = uw[:, t * _LANES:(t + 1) * _LANES]
        hi = jnp.maximum(m1, s)
        m1 = jnp.minimum(m1, s)
        m2 = jnp.minimum(m2, hi)

    v = jnp.min(jnp.minimum(m1, m2), axis=1, keepdims=True)
    for _ in range(_K):
        has1 = m1 == v
        m1 = jnp.where(has1, m2, m1)
        m2 = jnp.where(has1, jnp.inf, m2)
        v = jnp.min(jnp.minimum(m1, m2), axis=1, keepdims=True)

    eqf = (uw == v).astype(bf16)
    mm = (((1,), (0,)), ((), ()))
    a2p = jax.lax.dot_general(eqf, c0_s[...], mm,
                              preferred_element_type=f32)
    a2x = a2p[:, 0:1] + a2p[:, 2:3]
    a2y = a2p[:, 1:2] + a2p[:, 3:4]

    cntx = jnp.sum((ux < (a2x - sqx_b)).astype(f32), axis=1, keepdims=True)
    cnty = jnp.sum((uy < (a2y - sqy_b)).astype(f32), axis=1, keepdims=True)
    nx = jnp.where(a2x > _EPS, cntx, 0.0)
    ny = jnp.where(a2y > _EPS, cnty, 0.0)

    part = jnp.sum(jnp.log(nx + 1e-7) + jnp.log(ny + 1e-7))
    out_ref[...] = jnp.reshape(part, (1, 1, 1))


def _run(X, y, interpret=False):
    n, dx = X.shape
    dy = y.shape[1]
    grid = n // _BM
    parts = pl.pallas_call(
        _body,
        grid=(grid,),
        in_specs=[
            pl.BlockSpec((_BM, dx), lambda i: (i, 0)),
            pl.BlockSpec((_BM, dy), lambda i: (i, 0)),
            pl.BlockSpec((n, dx), lambda i: (0, 0)),
            pl.BlockSpec((n, dy), lambda i: (0, 0)),
        ],
        out_specs=pl.BlockSpec((1, 1, 1), lambda i: (i, 0, 0)),
        out_shape=jax.ShapeDtypeStruct((grid, 1, 1), jnp.float32),
        scratch_shapes=[
            pltpu.VMEM((n, 4), jnp.bfloat16),
            pltpu.VMEM((1, n), jnp.float32),
            pltpu.VMEM((1, n), jnp.float32),
        ],
        interpret=interpret,
    )(X, y, X, y)

    cx = math.pi ** (dx / 2.0) / math.gamma(dx / 2.0 + 1)
    cy = math.pi ** (dy / 2.0) / math.gamma(dy / 2.0 + 1)
    cxy = math.pi ** ((dx + dy) / 2.0) / math.gamma((dx + dy) / 2.0 + 1)
    c_log = math.log(cx * cy / cxy)
    digamma_k = -0.5772156649015329 + sum(1.0 / j for j in range(1, _K))
    n_avg_log = jnp.sum(parts) / jnp.float32(n)
    mi = (jnp.log(jnp.float32(n)) + jnp.float32(c_log)
          + jnp.float32(digamma_k) - n_avg_log) / jnp.log(jnp.float32(2.0))
    return jax.nn.relu(mi)


def kernel(X, y):
    return _run(X, y)
